# Initial kernel scaffold; baseline (speedup 1.0000x reference)
import jax
import jax.numpy as jnp
from jax.experimental import pallas as pl


def kernel(x, edge_index, W, b, prelu_a):
    raise NotImplementedError("write your pallas kernel here")


# speedup vs baseline: 38.6945x; 38.6945x over previous
import functools

import jax
import jax.numpy as jnp
from jax import lax
from jax.experimental import pallas as pl
from jax.experimental.pallas import tpu as pltpu
from jax.experimental.pallas import tpu_sc as plsc

N = 10000
NPAD = 10240
D = 128
E = 320000
CHUNK = 128
EROWS = 2560
EPAD = EROWS * CHUNK
ROWS_PER_TILE = EROWS // 32
NC, NS = 2, 16
STRIPE = NPAD // NS

_mesh = plsc.VectorSubcoreMesh(core_axis_name="c", subcore_axis_name="s")


@functools.partial(
    pl.kernel,
    out_type=jax.ShapeDtypeStruct((NC * NPAD,), jnp.float32),
    mesh=_mesh,
    scratch_types=[
        pltpu.VMEM((ROWS_PER_TILE, CHUNK), jnp.int32),
        pltpu.VMEM((CHUNK,), jnp.float32),
        pltpu.VMEM_SHARED((NPAD,), jnp.float32),
    ],
)
def _sc_degree(dst2d, zdeg, degp, idx_v, ones_v, acc):
    c = lax.axis_index("c")
    s = lax.axis_index("s")
    wid = s * NC + c

    @pl.when(s == 0)
    def _():
        pltpu.sync_copy(zdeg, acc)

    for k in range(CHUNK // 16):
        ones_v[pl.ds(k * 16, 16)] = jnp.ones((16,), jnp.float32)
    pltpu.sync_copy(dst2d.at[pl.ds(wid * ROWS_PER_TILE, ROWS_PER_TILE)], idx_v)
    plsc.subcore_barrier()

    def body(j, carry):
        pltpu.sync_copy(ones_v, acc.at[idx_v.at[j]], add=True)
        return carry

    lax.fori_loop(0, ROWS_PER_TILE, body, 0)
    plsc.subcore_barrier()
    pltpu.sync_copy(acc.at[pl.ds(s * STRIPE, STRIPE)],
                    degp.at[pl.ds(c * NPAD + s * STRIPE, STRIPE)])


def _tc_xw_body(x_ref, w_ref, d0_ref, d1_ref, y_ref, dinv_ref):
    deg = d0_ref[...] + d1_ref[...] + 1.0
    dinv = lax.rsqrt(deg)
    xw = jnp.dot(x_ref[...], w_ref[...], preferred_element_type=jnp.float32)
    y_ref[...] = xw * dinv
    dinv_ref[...] = dinv


IDXB = 16
NBLK = ROWS_PER_TILE // IDXB


@functools.partial(
    pl.kernel,
    out_type=jax.ShapeDtypeStruct((NC * NPAD, D), jnp.float32),
    mesh=_mesh,
    scratch_types=[
        pltpu.VMEM((IDXB, CHUNK), jnp.int32),
        pltpu.VMEM((IDXB, CHUNK), jnp.int32),
        pltpu.VMEM((CHUNK, D), jnp.float32),
        pltpu.VMEM((CHUNK, D), jnp.float32),
        pltpu.VMEM_SHARED((NPAD, D), jnp.float32),
        pltpu.SemaphoreType.DMA,
        pltpu.SemaphoreType.DMA,
    ],
)
def _sc_messages(y_hbm, src2d, dst2d, zbig, out_hbm,
                 sidx_v, didx_v, rows_a, rows_b, acc, gs0, gs1):
    c = lax.axis_index("c")
    s = lax.axis_index("s")
    wid = s * NC + c

    pltpu.sync_copy(zbig, acc.at[pl.ds(s * STRIPE, STRIPE)])
    plsc.subcore_barrier()

    for blk in range(NBLK):
        base = wid * ROWS_PER_TILE + blk * IDXB
        pltpu.sync_copy(src2d.at[pl.ds(base, IDXB)], sidx_v)
        pltpu.sync_copy(dst2d.at[pl.ds(base, IDXB)], didx_v)
        pltpu.async_copy(y_hbm.at[sidx_v.at[0]], rows_a, gs0)
        pltpu.async_copy(y_hbm.at[sidx_v.at[1]], rows_b, gs1)

        def body(i, carry):
            j0 = 2 * i
            pltpu.make_async_copy(y_hbm.at[sidx_v.at[j0]], rows_a, gs0).wait()
            pltpu.sync_copy(rows_a, acc.at[didx_v.at[j0]], add=True)
            pltpu.async_copy(y_hbm.at[sidx_v.at[j0 + 2]], rows_a, gs0)
            pltpu.make_async_copy(y_hbm.at[sidx_v.at[j0 + 1]], rows_b, gs1).wait()
            pltpu.sync_copy(rows_b, acc.at[didx_v.at[j0 + 1]], add=True)
            pltpu.async_copy(y_hbm.at[sidx_v.at[j0 + 3]], rows_b, gs1)
            return carry

        lax.fori_loop(0, IDXB // 2 - 1, body, 0)
        j0 = IDXB - 2
        pltpu.make_async_copy(y_hbm.at[sidx_v.at[j0]], rows_a, gs0).wait()
        pltpu.sync_copy(rows_a, acc.at[didx_v.at[j0]], add=True)
        pltpu.make_async_copy(y_hbm.at[sidx_v.at[j0 + 1]], rows_b, gs1).wait()
        pltpu.sync_copy(rows_b, acc.at[didx_v.at[j0 + 1]], add=True)

    plsc.subcore_barrier()
    pltpu.sync_copy(acc.at[pl.ds(s * STRIPE, STRIPE)],
                    out_hbm.at[pl.ds(c * NPAD + s * STRIPE, STRIPE)])


def _tc_out_body(p_ref, y_ref, dinv_ref, b_ref, a_ref, o_ref):
    pp = p_ref[...]
    t = (pp[0] + pp[1] + y_ref[...]) * dinv_ref[...] + b_ref[...]
    a = a_ref[0, 0]
    o_ref[...] = jnp.where(t >= 0, t, a * t)


def kernel(x, edge_index, W, b, prelu_a):
    src = edge_index[0]
    dst = edge_index[1]
    npad = EPAD - E
    fill = jnp.arange(npad, dtype=jnp.int32)
    src_p = jnp.concatenate([src, fill % N]).reshape(EROWS, CHUNK)
    dst_p = jnp.concatenate([dst, N + (fill % (NPAD - N))]).reshape(EROWS, CHUNK)

    zdeg = jnp.zeros((NPAD,), jnp.float32)
    zbig = jnp.zeros((STRIPE, D), jnp.float32)

    degp = _sc_degree(dst_p, zdeg)
    d0 = degp[:N].reshape(N, 1)
    d1 = degp[NPAD:NPAD + N].reshape(N, 1)

    RB = 1000
    grid = N // RB
    y, dinv = pl.pallas_call(
        _tc_xw_body,
        grid=(grid,),
        in_specs=[
            pl.BlockSpec((RB, D), lambda i: (i, 0)),
            pl.BlockSpec((D, D), lambda i: (0, 0)),
            pl.BlockSpec((RB, 1), lambda i: (i, 0)),
            pl.BlockSpec((RB, 1), lambda i: (i, 0)),
        ],
        out_specs=[
            pl.BlockSpec((RB, D), lambda i: (i, 0)),
            pl.BlockSpec((RB, 1), lambda i: (i, 0)),
        ],
        out_shape=[
            jax.ShapeDtypeStruct((N, D), jnp.float32),
            jax.ShapeDtypeStruct((N, 1), jnp.float32),
        ],
    )(x, W, d0, d1)

    p = _sc_messages(y, src_p, dst_p, zbig).reshape(NC, NPAD, D)

    out = pl.pallas_call(
        _tc_out_body,
        grid=(grid,),
        in_specs=[
            pl.BlockSpec((NC, RB, D), lambda i: (0, i, 0)),
            pl.BlockSpec((RB, D), lambda i: (i, 0)),
            pl.BlockSpec((RB, 1), lambda i: (i, 0)),
            pl.BlockSpec((1, D), lambda i: (0, 0)),
            pl.BlockSpec((1, 1), lambda i: (0, 0)),
        ],
        out_specs=pl.BlockSpec((RB, D), lambda i: (i, 0)),
        out_shape=jax.ShapeDtypeStruct((N, D), jnp.float32),
    )(p, y, dinv, b.reshape(1, D), prelu_a.reshape(1, 1))
    return out
